# Initial kernel scaffold; baseline (speedup 1.0000x reference)
import jax
import jax.numpy as jnp
from jax.experimental import pallas as pl


def kernel(x, edge_index, W1, b1, W2, b2, W3, b3, Wp, bp):
    raise NotImplementedError("write your pallas kernel here")


# speedup vs baseline: 9.9153x; 9.9153x over previous
import functools

import jax
import jax.numpy as jnp
from jax import lax
from jax.experimental import pallas as pl
from jax.experimental.pallas import tpu as pltpu
from jax.experimental.pallas import tpu_sc as plsc

N = 10000
E = 320000
NC, NS = 2, 16
NW = NC * NS
EW = E // NW
CH = 80
NCHUNK = EW // CH
NP = 10240
RPT = NP // NS
DEGW = 128

_sc_mesh = plsc.VectorSubcoreMesh(
    core_axis_name="c", subcore_axis_name="s", num_cores=NC, num_subcores=NS)


@functools.partial(
    pl.kernel,
    out_type=jax.ShapeDtypeStruct((NC, NP, DEGW), jnp.float32),
    mesh=_sc_mesh,
    scratch_types=[
        pltpu.VMEM((CH,), jnp.int32),
        pltpu.VMEM((CH, DEGW), jnp.float32),
        pltpu.VMEM_SHARED((NP, DEGW), jnp.float32),
        pltpu.SemaphoreType.DMA,
    ],
)
def _sc_degree(dst_hbm, ones_hbm, zeros_hbm, out_hbm, didx, ones_v, acc, sem):
    cid = lax.axis_index("c")
    sid = lax.axis_index("s")
    base = (cid * NS + sid) * EW
    r0 = pl.multiple_of(sid * RPT, 8)
    pltpu.sync_copy(zeros_hbm.at[pl.ds(r0, RPT)], acc.at[pl.ds(r0, RPT)])
    pltpu.sync_copy(ones_hbm, ones_v)
    plsc.subcore_barrier()

    def body(i, carry):
        off = pl.multiple_of(base + i * CH, 8)
        pltpu.sync_copy(dst_hbm.at[pl.ds(off, CH)], didx)
        pltpu.sync_copy(ones_v, acc.at[didx], add=True)
        return carry

    lax.fori_loop(0, NCHUNK, body, 0)
    plsc.subcore_barrier()
    pltpu.sync_copy(acc.at[pl.ds(r0, RPT)], out_hbm.at[cid, pl.ds(r0, RPT)])


@functools.partial(
    pl.kernel,
    out_type=jax.ShapeDtypeStruct((NC, NP, 128), jnp.float32),
    mesh=_sc_mesh,
    scratch_types=[
        pltpu.VMEM((CH,), jnp.int32),
        pltpu.VMEM((CH,), jnp.int32),
        pltpu.VMEM((CH, 128), jnp.float32),
        pltpu.VMEM_SHARED((NP, 128), jnp.float32),
        pltpu.SemaphoreType.DMA,
    ],
)
def _sc_spass(src_hbm, dst_hbm, ytab_hbm, zeros_hbm, out_hbm,
              sidx, didx, rows, acc, sem):
    cid = lax.axis_index("c")
    sid = lax.axis_index("s")
    base = (cid * NS + sid) * EW
    r0 = pl.multiple_of(sid * RPT, 8)
    pltpu.sync_copy(zeros_hbm.at[pl.ds(r0, RPT)], acc.at[pl.ds(r0, RPT)])
    plsc.subcore_barrier()

    def body(i, carry):
        off = pl.multiple_of(base + i * CH, 8)
        pltpu.sync_copy(src_hbm.at[pl.ds(off, CH)], sidx)
        pltpu.sync_copy(dst_hbm.at[pl.ds(off, CH)], didx)
        pltpu.async_copy(ytab_hbm.at[sidx], rows, sem).wait()
        pltpu.sync_copy(rows, acc.at[didx], add=True)
        return carry

    lax.fori_loop(0, NCHUNK, body, 0)
    plsc.subcore_barrier()
    pltpu.sync_copy(acc.at[pl.ds(r0, RPT)], out_hbm.at[cid, pl.ds(r0, RPT)])


ROWS = 1000


def _tc1_body(degp_ref, x_ref, dinv_ref, y0_ref):
    deg = 1.0 + degp_ref[0, :, 0:1] + degp_ref[1, :, 0:1]
    dinv = lax.rsqrt(deg)
    dinv_ref[...] = dinv
    y0_ref[...] = x_ref[...] * dinv


def _tc2_body(p_ref, y0_ref, dinv_ref, w1_ref, b1_ref, y1a_ref, y1b_ref):
    dinv = dinv_ref[...]
    agg = (p_ref[0] + p_ref[1] + y0_ref[...]) * dinv
    h1 = jnp.dot(agg, w1_ref[...], preferred_element_type=jnp.float32)
    h1 = jnp.maximum(h1 + b1_ref[...], 0.0)
    y1 = h1 * dinv
    y1a_ref[...] = y1[:, :128]
    y1b_ref[...] = y1[:, 128:]


def _tc3_body(pa_ref, pb_ref, y1a_ref, y1b_ref, dinv_ref, w2_ref, b2_ref,
              w3_ref, y2_ref):
    dinv = dinv_ref[...]
    agg_a = (pa_ref[0] + pa_ref[1] + y1a_ref[...]) * dinv
    agg_b = (pb_ref[0] + pb_ref[1] + y1b_ref[...]) * dinv
    agg = jnp.concatenate([agg_a, agg_b], axis=1)
    h2 = jnp.dot(agg, w2_ref[...], preferred_element_type=jnp.float32)
    h2 = jnp.maximum(h2 + b2_ref[...], 0.0)
    g = jnp.dot(h2, w3_ref[...], preferred_element_type=jnp.float32)
    y2_ref[...] = g * dinv


def _tc4_body(p_ref, y2_ref, dinv_ref, b3_ref, wp_ref, bp_ref, out_ref):
    agg = (p_ref[0] + p_ref[1] + y2_ref[...]) * dinv_ref[...]
    h3 = jnp.maximum(agg + b3_ref[...], 0.0)
    out = jnp.dot(h3, wp_ref[...], preferred_element_type=jnp.float32)
    out_ref[...] = out + bp_ref[...]


def _rowblk(d):
    return pl.BlockSpec((ROWS, d), lambda i: (i, 0))


def _partblk(d):
    return pl.BlockSpec((NC, ROWS, d), lambda i: (0, i, 0))


def _full(a, b):
    return pl.BlockSpec((a, b), lambda i: (0, 0))


def _tc_call(body, in_specs, out_specs, out_shape, *args):
    return pl.pallas_call(
        body,
        grid=(N // ROWS,),
        in_specs=in_specs,
        out_specs=out_specs,
        out_shape=out_shape,
    )(*args)


@jax.jit
def kernel(x, edge_index, W1, b1, W2, b2, W3, b3, Wp, bp):
    f32 = jnp.float32
    src = edge_index[0]
    dst = edge_index[1]
    zeros128 = jnp.zeros((NP, 128), f32)
    zeros16 = jnp.zeros((NP, DEGW), f32)
    ones16 = jnp.ones((CH, DEGW), f32)
    b1r, b2r = b1.reshape(1, -1), b2.reshape(1, -1)
    b3r, bpr = b3.reshape(1, -1), bp.reshape(1, -1)

    degp = _sc_degree(dst, ones16, zeros16)

    dinv, y0 = _tc_call(
        _tc1_body,
        [_partblk(DEGW), _rowblk(128)],
        [_rowblk(1), _rowblk(128)],
        (jax.ShapeDtypeStruct((N, 1), f32), jax.ShapeDtypeStruct((N, 128), f32)),
        degp, x)

    p1 = _sc_spass(src, dst, y0, zeros128)

    y1a, y1b = _tc_call(
        _tc2_body,
        [_partblk(128), _rowblk(128), _rowblk(1), _full(128, 256), _full(1, 256)],
        [_rowblk(128), _rowblk(128)],
        (jax.ShapeDtypeStruct((N, 128), f32), jax.ShapeDtypeStruct((N, 128), f32)),
        p1, y0, dinv, W1, b1r)

    p2a = _sc_spass(src, dst, y1a, zeros128)
    p2b = _sc_spass(src, dst, y1b, zeros128)

    y2 = _tc_call(
        _tc3_body,
        [_partblk(128), _partblk(128), _rowblk(128), _rowblk(128), _rowblk(1),
         _full(256, 256), _full(1, 256), _full(256, 128)],
        _rowblk(128),
        jax.ShapeDtypeStruct((N, 128), f32),
        p2a, p2b, y1a, y1b, dinv, W2, b2r, W3)

    p3 = _sc_spass(src, dst, y2, zeros128)

    out = _tc_call(
        _tc4_body,
        [_partblk(128), _rowblk(128), _rowblk(1), _full(1, 128),
         _full(128, 128), _full(1, 128)],
        _rowblk(128),
        jax.ShapeDtypeStruct((N, 128), f32),
        p3, y2, dinv, b3r, Wp, bpr)

    return out


# speedup vs baseline: 22.4265x; 2.2618x over previous
import functools

import jax
import jax.numpy as jnp
from jax import lax
from jax.experimental import pallas as pl
from jax.experimental.pallas import tpu as pltpu
from jax.experimental.pallas import tpu_sc as plsc

N = 10000
E = 320000
NC, NS = 2, 16
NW = NC * NS
EW = E // NW
CH = 80
NCHUNK = EW // CH
NP = 10240
RPT = NP // NS
DEGW = 128

_sc_mesh = plsc.VectorSubcoreMesh(
    core_axis_name="c", subcore_axis_name="s", num_cores=NC, num_subcores=NS)


@functools.partial(
    pl.kernel,
    out_type=jax.ShapeDtypeStruct((NC, NP, DEGW), jnp.float32),
    mesh=_sc_mesh,
    scratch_types=[
        pltpu.VMEM((NCHUNK, CH), jnp.int32),
        pltpu.VMEM((CH, DEGW), jnp.float32),
        pltpu.VMEM_SHARED((NP, DEGW), jnp.float32),
        pltpu.SemaphoreType.DMA,
    ],
)
def _sc_degree(dst_hbm, ones_hbm, zeros_hbm, out_hbm, didx, ones_v, acc, sem):
    cid = lax.axis_index("c")
    sid = lax.axis_index("s")
    wid = cid * NS + sid
    r0 = pl.multiple_of(sid * RPT, 8)
    pltpu.sync_copy(zeros_hbm.at[pl.ds(r0, RPT)], acc.at[pl.ds(r0, RPT)])
    pltpu.sync_copy(ones_hbm, ones_v)
    pltpu.sync_copy(dst_hbm.at[wid], didx)
    plsc.subcore_barrier()

    def body(i, carry):
        pltpu.sync_copy(ones_v, acc.at[didx.at[i]], add=True)
        return carry

    lax.fori_loop(0, NCHUNK, body, 0)
    plsc.subcore_barrier()
    pltpu.sync_copy(acc.at[pl.ds(r0, RPT)], out_hbm.at[cid, pl.ds(r0, RPT)])


@functools.partial(
    pl.kernel,
    out_type=jax.ShapeDtypeStruct((NC, NP, 128), jnp.float32),
    mesh=_sc_mesh,
    scratch_types=[
        pltpu.VMEM((EW,), jnp.int32),
        pltpu.VMEM((NCHUNK, CH), jnp.int32),
        pltpu.VMEM((CH, 128), jnp.float32),
        pltpu.VMEM((CH, 128), jnp.float32),
        pltpu.VMEM_SHARED((NP, 128), jnp.float32),
        pltpu.SemaphoreType.DMA,
        pltpu.SemaphoreType.DMA,
    ],
)
def _sc_spass(src_hbm, dst_hbm, ytab_hbm, zeros_hbm, out_hbm,
              sidx, didx, rows0, rows1, acc, sem0, sem1):
    cid = lax.axis_index("c")
    sid = lax.axis_index("s")
    wid = cid * NS + sid
    r0 = pl.multiple_of(sid * RPT, 8)
    pltpu.sync_copy(zeros_hbm.at[pl.ds(r0, RPT)], acc.at[pl.ds(r0, RPT)])
    pltpu.sync_copy(src_hbm.at[pl.ds(wid * EW, EW)], sidx)
    pltpu.sync_copy(dst_hbm.at[wid], didx)
    plsc.subcore_barrier()

    def sgather(c, buf, sem):
        off = pl.multiple_of(c * CH, 8)
        return pltpu.async_copy(ytab_hbm.at[sidx.at[pl.ds(off, CH)]], buf, sem)

    sgather(0, rows0, sem0)

    def pair(j, carry):
        c0 = j * 2
        sgather(c0 + 1, rows1, sem1)
        pltpu.make_async_copy(ytab_hbm.at[sidx.at[pl.ds(0, CH)]], rows0, sem0).wait()
        pltpu.sync_copy(rows0, acc.at[didx.at[c0]], add=True)
        sgather(c0 + 2, rows0, sem0)
        pltpu.make_async_copy(ytab_hbm.at[sidx.at[pl.ds(0, CH)]], rows1, sem1).wait()
        pltpu.sync_copy(rows1, acc.at[didx.at[c0 + 1]], add=True)
        return carry

    lax.fori_loop(0, (NCHUNK - 1) // 2, pair, 0)
    pltpu.make_async_copy(ytab_hbm.at[sidx.at[pl.ds(0, CH)]], rows0, sem0).wait()
    pltpu.sync_copy(rows0, acc.at[didx.at[NCHUNK - 1]], add=True)

    plsc.subcore_barrier()
    pltpu.sync_copy(acc.at[pl.ds(r0, RPT)], out_hbm.at[cid, pl.ds(r0, RPT)])


ROWS = 1000


def _tc1_body(degp_ref, x_ref, dinv_ref, y0_ref):
    deg = 1.0 + degp_ref[0, :, 0:1] + degp_ref[1, :, 0:1]
    dinv = lax.rsqrt(deg)
    dinv_ref[...] = dinv
    y0_ref[...] = x_ref[...] * dinv


def _tc2_body(p_ref, y0_ref, dinv_ref, w1_ref, b1_ref, y1a_ref, y1b_ref):
    dinv = dinv_ref[...]
    agg = (p_ref[0] + p_ref[1] + y0_ref[...]) * dinv
    h1 = jnp.dot(agg, w1_ref[...], preferred_element_type=jnp.float32)
    h1 = jnp.maximum(h1 + b1_ref[...], 0.0)
    y1 = h1 * dinv
    y1a_ref[...] = y1[:, :128]
    y1b_ref[...] = y1[:, 128:]


def _tc3_body(pa_ref, pb_ref, y1a_ref, y1b_ref, dinv_ref, w2_ref, b2_ref,
              w3_ref, y2_ref):
    dinv = dinv_ref[...]
    agg_a = (pa_ref[0] + pa_ref[1] + y1a_ref[...]) * dinv
    agg_b = (pb_ref[0] + pb_ref[1] + y1b_ref[...]) * dinv
    agg = jnp.concatenate([agg_a, agg_b], axis=1)
    h2 = jnp.dot(agg, w2_ref[...], preferred_element_type=jnp.float32)
    h2 = jnp.maximum(h2 + b2_ref[...], 0.0)
    g = jnp.dot(h2, w3_ref[...], preferred_element_type=jnp.float32)
    y2_ref[...] = g * dinv


def _tc4_body(p_ref, y2_ref, dinv_ref, b3_ref, wp_ref, bp_ref, out_ref):
    agg = (p_ref[0] + p_ref[1] + y2_ref[...]) * dinv_ref[...]
    h3 = jnp.maximum(agg + b3_ref[...], 0.0)
    out = jnp.dot(h3, wp_ref[...], preferred_element_type=jnp.float32)
    out_ref[...] = out + bp_ref[...]


def _rowblk(d):
    return pl.BlockSpec((ROWS, d), lambda i: (i, 0))


def _partblk(d):
    return pl.BlockSpec((NC, ROWS, d), lambda i: (0, i, 0))


def _full(a, b):
    return pl.BlockSpec((a, b), lambda i: (0, 0))


def _tc_call(body, in_specs, out_specs, out_shape, *args):
    return pl.pallas_call(
        body,
        grid=(N // ROWS,),
        in_specs=in_specs,
        out_specs=out_specs,
        out_shape=out_shape,
    )(*args)


@jax.jit
def kernel(x, edge_index, W1, b1, W2, b2, W3, b3, Wp, bp):
    f32 = jnp.float32
    src = edge_index[0]
    dst = edge_index[1].reshape(NW, NCHUNK, CH)
    zeros128 = jnp.zeros((NP, 128), f32)
    zeros16 = jnp.zeros((NP, DEGW), f32)
    ones16 = jnp.ones((CH, DEGW), f32)
    b1r, b2r = b1.reshape(1, -1), b2.reshape(1, -1)
    b3r, bpr = b3.reshape(1, -1), bp.reshape(1, -1)

    degp = _sc_degree(dst, ones16, zeros16)

    dinv, y0 = _tc_call(
        _tc1_body,
        [_partblk(DEGW), _rowblk(128)],
        [_rowblk(1), _rowblk(128)],
        (jax.ShapeDtypeStruct((N, 1), f32), jax.ShapeDtypeStruct((N, 128), f32)),
        degp, x)

    p1 = _sc_spass(src, dst, y0, zeros128)

    y1a, y1b = _tc_call(
        _tc2_body,
        [_partblk(128), _rowblk(128), _rowblk(1), _full(128, 256), _full(1, 256)],
        [_rowblk(128), _rowblk(128)],
        (jax.ShapeDtypeStruct((N, 128), f32), jax.ShapeDtypeStruct((N, 128), f32)),
        p1, y0, dinv, W1, b1r)

    p2a = _sc_spass(src, dst, y1a, zeros128)
    p2b = _sc_spass(src, dst, y1b, zeros128)

    y2 = _tc_call(
        _tc3_body,
        [_partblk(128), _partblk(128), _rowblk(128), _rowblk(128), _rowblk(1),
         _full(256, 256), _full(1, 256), _full(256, 128)],
        _rowblk(128),
        jax.ShapeDtypeStruct((N, 128), f32),
        p2a, p2b, y1a, y1b, dinv, W2, b2r, W3)

    p3 = _sc_spass(src, dst, y2, zeros128)

    out = _tc_call(
        _tc4_body,
        [_partblk(128), _rowblk(128), _rowblk(1), _full(1, 128),
         _full(128, 128), _full(1, 128)],
        _rowblk(128),
        jax.ShapeDtypeStruct((N, 128), f32),
        p3, y2, dinv, b3r, Wp, bpr)

    return out
